# Initial kernel scaffold; baseline (speedup 1.0000x reference)
import jax
import jax.numpy as jnp
from jax.experimental import pallas as pl


def kernel(user_emb, item_emb, edge_weight, edge_index):
    raise NotImplementedError("write your pallas kernel here")


# speedup vs baseline: 3.1414x; 3.1414x over previous
import functools

import jax
import jax.numpy as jnp
from jax import lax
from jax.experimental import pallas as pl
from jax.experimental.pallas import tpu as pltpu
from jax.experimental.pallas import tpu_sc as plsc

NC = 2
NS = 16
C = 128
DH = 32
ZR = 136


def _layer_body(n_pad, ept, emb_hbm, src_hbm, dst_hbm, w_hbm, out_hbm,
                idx_s, idx_d, wts, rows, zbuf, acc, sem):
    c = lax.axis_index("c")
    s = lax.axis_index("s")
    rows_per_tile = n_pad // NS

    def zfill(r, carry):
        zbuf[r, 0:16] = jnp.zeros((16,), jnp.float32)
        zbuf[r, 16:32] = jnp.zeros((16,), jnp.float32)
        return carry
    lax.fori_loop(0, ZR, zfill, 0)
    def zdma(k, carry):
        pltpu.sync_copy(zbuf,
                        acc.at[pl.ds(s * rows_per_tile + k * ZR, ZR)])
        return carry
    lax.fori_loop(0, rows_per_tile // ZR, zdma, 0)
    plsc.subcore_barrier()

    base = s * ept
    def chunk(i, carry):
        off = base + i * C
        pltpu.sync_copy(src_hbm.at[pl.ds(off, C)], idx_s)
        pltpu.sync_copy(dst_hbm.at[pl.ds(off, C)], idx_d)
        pltpu.sync_copy(w_hbm.at[pl.ds(off, C)], wts)
        pltpu.async_copy(emb_hbm.at[c].at[idx_s], rows, sem).wait()
        def scale16(j, carry2):
            wv = wts[pl.ds(j * 16, 16)]
            for k in range(16):
                e = j * 16 + k
                rows[e, 0:16] = rows[e, 0:16] * wv[k]
                rows[e, 16:32] = rows[e, 16:32] * wv[k]
            return carry2
        lax.fori_loop(0, C // 16, scale16, 0)
        pltpu.sync_copy(rows, acc.at[idx_d], add=True)
        return carry
    lax.fori_loop(0, ept // C, chunk, 0)
    plsc.subcore_barrier()

    pltpu.sync_copy(acc.at[pl.ds(s * rows_per_tile, rows_per_tile)],
                    out_hbm.at[c].at[pl.ds(s * rows_per_tile, rows_per_tile)])


@functools.partial(jax.jit, static_argnums=(4, 5))
def _layer(emb2, src, dst, w, n_pad, ept):
    mesh = plsc.VectorSubcoreMesh(core_axis_name="c", subcore_axis_name="s")
    body = functools.partial(_layer_body, n_pad, ept)
    return pl.kernel(
        body,
        out_type=jax.ShapeDtypeStruct((NC, n_pad, DH), jnp.float32),
        mesh=mesh,
        compiler_params=pltpu.CompilerParams(use_tc_tiling_on_sc=False),
        scratch_types=[
            pltpu.VMEM((C,), jnp.int32),
            pltpu.VMEM((C,), jnp.int32),
            pltpu.VMEM((C,), jnp.float32),
            pltpu.VMEM((C, DH), jnp.float32),
            pltpu.VMEM((ZR, DH), jnp.float32),
            pltpu.VMEM_SHARED((n_pad, DH), jnp.float32),
            pltpu.SemaphoreType.DMA,
        ],
    )(emb2, src, dst, w)


def kernel(user_emb, item_emb, edge_weight, edge_index):
    n_users = user_emb.shape[0]
    n_nodes = n_users + item_emb.shape[0]
    e = edge_weight.shape[0]

    blk_n = NS * ZR
    n_pad = ((n_nodes + blk_n - 1) // blk_n) * blk_n

    all_emb = jnp.concatenate([user_emb, item_emb], axis=0)
    emb2 = all_emb.reshape(n_nodes, NC, DH).transpose(1, 0, 2)
    emb2 = jnp.pad(emb2, ((0, 0), (0, n_pad - n_nodes), (0, 0)))

    blk_e = NS * C
    e_pad = ((e + blk_e - 1) // blk_e) * blk_e
    src = edge_index[0]
    dst = edge_index[1]
    w = edge_weight
    if e_pad != e:
        pad = e_pad - e
        src = jnp.concatenate([src, jnp.zeros((pad,), src.dtype)])
        dst = jnp.concatenate([dst, jnp.zeros((pad,), dst.dtype)])
        w = jnp.concatenate([w, jnp.zeros((pad,), w.dtype)])
    ept = e_pad // NS

    acc_sum = emb2
    cur = emb2
    for _ in range(3):
        cur = _layer(cur, src, dst, w, n_pad, ept)
        acc_sum = acc_sum + cur

    final = (acc_sum * 0.25).transpose(1, 0, 2).reshape(n_pad, NC * DH)
    return (final[:n_users], final[n_users:n_nodes])


# speedup vs baseline: 8.9955x; 2.8635x over previous
import functools

import jax
import jax.numpy as jnp
from jax import lax
from jax.experimental import pallas as pl
from jax.experimental.pallas import tpu as pltpu
from jax.experimental.pallas import tpu_sc as plsc

NC = 2
NS = 16
C = 128
G = 2
SU = C * G
DH = 32
ZR = 136


def _layer_body(n_pad, scpt, emb_hbm, idx_hbm, w_hbm, out_hbm,
                ebuf0, ebuf1, wbuf0, wbuf1, rows0, rows1, acc,
                e_sem0, e_sem1, g_sem0, g_sem1, s_sem):
    c = lax.axis_index("c")
    s = lax.axis_index("s")
    rows_per_tile = n_pad // NS
    ebuf = (ebuf0, ebuf1)
    wbuf = (wbuf0, wbuf1)
    rows = (rows0, rows1)
    e_sem = (e_sem0, e_sem1)
    g_sem = (g_sem0, g_sem1)
    total = scpt

    def zfill(r, carry):
        rows0[r, 0:16] = jnp.zeros((16,), jnp.float32)
        rows0[r, 16:32] = jnp.zeros((16,), jnp.float32)
        return carry
    lax.fori_loop(0, ZR, zfill, 0)
    def zdma(k, carry):
        pltpu.sync_copy(rows0.at[pl.ds(0, ZR)],
                        acc.at[pl.ds(s * rows_per_tile + k * ZR, ZR)])
        return carry
    lax.fori_loop(0, rows_per_tile // ZR, zdma, 0)
    plsc.subcore_barrier()

    emb_c = emb_hbm.at[c]

    def issue_edge_dma(t, p):
        pltpu.async_copy(idx_hbm.at[t], ebuf[p], e_sem[p])
        pltpu.async_copy(w_hbm.at[t], wbuf[p], e_sem[p])

    def wait_edge_dma(t, p):
        pltpu.make_async_copy(idx_hbm.at[t], ebuf[p], e_sem[p]).wait()
        pltpu.make_async_copy(w_hbm.at[t], wbuf[p], e_sem[p]).wait()

    def issue_gathers(p):
        for g in range(G):
            pltpu.async_copy(emb_c.at[ebuf[p].at[g]],
                             rows[p].at[pl.ds(g * C, C)], g_sem[p])

    def wait_gathers(p):
        for g in range(G):
            pltpu.make_async_copy(emb_c.at[ebuf[p].at[g]],
                                  rows[p].at[pl.ds(g * C, C)],
                                  g_sem[p]).wait()

    base_t = s * scpt
    issue_edge_dma(base_t, 0)
    issue_edge_dma(base_t + 1, 1)
    wait_edge_dma(base_t, 0)
    issue_gathers(0)

    def pair(i, carry):
        for p in (0, 1):
            q = 1 - p
            sc = 2 * i + p
            @pl.when(sc < total - 1)
            def _():
                wait_edge_dma(base_t + sc + 1, q)
                issue_gathers(q)
            wait_gathers(p)

            def scale(j, carry2):
                wv = wbuf[p][pl.ds(j * 16, 16)]
                for k in range(16):
                    e = j * 16 + k
                    rows[p][e, 0:16] = rows[p][e, 0:16] * wv[k]
                    rows[p][e, 16:32] = rows[p][e, 16:32] * wv[k]
                return carry2
            lax.fori_loop(0, SU // 16, scale, 0)

            descs = []
            for g in range(G):
                descs.append(pltpu.async_copy(
                    rows[p].at[pl.ds(g * C, C)],
                    acc.at[ebuf[p].at[G + g]], s_sem, add=True))
            for d in descs:
                d.wait()

            @pl.when(sc < total - 2)
            def _():
                issue_edge_dma(base_t + sc + 2, p)
        return carry
    lax.fori_loop(0, total // 2, pair, 0)
    plsc.subcore_barrier()

    pltpu.sync_copy(acc.at[pl.ds(s * rows_per_tile, rows_per_tile)],
                    out_hbm.at[c].at[pl.ds(s * rows_per_tile, rows_per_tile)])


@functools.partial(jax.jit, static_argnums=(3, 4))
def _layer(emb2, idx_packed, w_packed, n_pad, scpt):
    mesh = plsc.VectorSubcoreMesh(core_axis_name="c", subcore_axis_name="s")
    body = functools.partial(_layer_body, n_pad, scpt)
    return pl.kernel(
        body,
        out_type=jax.ShapeDtypeStruct((NC, n_pad, DH), jnp.float32),
        mesh=mesh,
        compiler_params=pltpu.CompilerParams(use_tc_tiling_on_sc=False),
        scratch_types=[
            pltpu.VMEM((2 * G, C), jnp.int32),
            pltpu.VMEM((2 * G, C), jnp.int32),
            pltpu.VMEM((SU,), jnp.float32),
            pltpu.VMEM((SU,), jnp.float32),
            pltpu.VMEM((SU, DH), jnp.float32),
            pltpu.VMEM((SU, DH), jnp.float32),
            pltpu.VMEM_SHARED((n_pad, DH), jnp.float32),
            pltpu.SemaphoreType.DMA,
            pltpu.SemaphoreType.DMA,
            pltpu.SemaphoreType.DMA,
            pltpu.SemaphoreType.DMA,
            pltpu.SemaphoreType.DMA,
        ],
    )(emb2, idx_packed, w_packed)


def kernel(user_emb, item_emb, edge_weight, edge_index):
    n_users = user_emb.shape[0]
    n_nodes = n_users + item_emb.shape[0]
    e = edge_weight.shape[0]

    blk_n = NS * ZR
    n_pad = ((n_nodes + blk_n - 1) // blk_n) * blk_n

    all_emb = jnp.concatenate([user_emb, item_emb], axis=0)
    emb2 = all_emb.reshape(n_nodes, NC, DH).transpose(1, 0, 2)
    emb2 = jnp.pad(emb2, ((0, 0), (0, n_pad - n_nodes), (0, 0)))

    blk_e = NS * SU * 2
    e_pad = ((e + blk_e - 1) // blk_e) * blk_e
    src = edge_index[0]
    dst = edge_index[1]
    w = edge_weight
    if e_pad != e:
        pad = e_pad - e
        src = jnp.concatenate([src, jnp.zeros((pad,), src.dtype)])
        dst = jnp.concatenate([dst, jnp.zeros((pad,), dst.dtype)])
        w = jnp.concatenate([w, jnp.zeros((pad,), w.dtype)])
    n_super = e_pad // SU
    scpt = n_super // NS
    src3 = src.reshape(n_super, G, C)
    dst3 = dst.reshape(n_super, G, C)
    idx_packed = jnp.concatenate([src3, dst3], axis=1)
    w_packed = w.reshape(n_super, SU)

    acc_sum = emb2
    cur = emb2
    for _ in range(3):
        cur = _layer(cur, idx_packed, w_packed, n_pad, scpt)
        acc_sum = acc_sum + cur

    final = (acc_sum * 0.25).transpose(1, 0, 2).reshape(n_pad, NC * DH)
    return (final[:n_users], final[n_users:n_nodes])
